# baseline (speedup 1.0000x reference)
import functools

import jax
import jax.numpy as jnp
from jax import lax
from jax.experimental import pallas as pl
from jax.experimental.pallas import tpu as pltpu
from jax.experimental.pallas import tpu_sc as plsc

B = 4096
K = 32

_info = plsc.get_sparse_core_info()
_NC, _NS = _info.num_cores, _info.num_subcores
_NW = _NC * _NS
_BPW = B // _NW
_NG = _BPW // 16


def _splat(x, n=16):
    return jnp.full((n,), x, jnp.int32)


def _sc_user_body(user_idx, utab_t, out_ut, idx_u, buf_u, cols_u, sem_u):
    wid = lax.axis_index("s") * _NC + lax.axis_index("c")
    base = wid * _BPW
    pltpu.sync_copy(user_idx.at[pl.ds(base, _BPW)], idx_u)
    c_lo = lax.iota(jnp.int32, 16)
    c_hi = c_lo + 16

    def group(h, carry):
        vu = idx_u[pl.ds(h * 16, 16)]
        tu = lax.shift_right_logical(vu, 7)
        ru = lax.bitwise_and(vu, _splat(127))
        for half in range(2):
            for k in range(8):
                lane = 8 * half + k
                offu = pl.multiple_of(tu[lane] * 128, 128)
                pltpu.make_async_copy(
                    utab_t.at[:, pl.ds(offu, 128)], buf_u.at[k], sem_u).start()
            for k in range(8):
                pltpu.make_async_copy(
                    utab_t.at[:, pl.ds(0, 128)], buf_u.at[k], sem_u).wait()
            for k in range(8):
                lane = 8 * half + k
                slot = _splat(k)
                jv = _splat(h * 16 + lane)
                rmu = _splat(ru[lane])
                u_lo = plsc.load_gather(buf_u, [slot, c_lo, rmu])
                u_hi = plsc.load_gather(buf_u, [slot, c_hi, rmu])
                plsc.store_scatter(cols_u, [c_lo, jv], u_lo)
                plsc.store_scatter(cols_u, [c_hi, jv], u_hi)
        return carry

    lax.fori_loop(0, _NG, group, 0)
    pltpu.sync_copy(cols_u, out_ut.at[:, pl.ds(base, _BPW)])


_sc_user = functools.partial(
    pl.kernel,
    mesh=plsc.VectorSubcoreMesh(core_axis_name="c", subcore_axis_name="s"),
    out_type=jax.ShapeDtypeStruct((K, B), jnp.float32),
    scratch_types=[
        pltpu.VMEM((_BPW,), jnp.int32),
        pltpu.VMEM((8, K, 128), jnp.float32),
        pltpu.VMEM((K, _BPW), jnp.float32),
        pltpu.SemaphoreType.DMA,
    ],
    compiler_params=pltpu.CompilerParams(needs_layout_passes=False),
)(_sc_user_body)


def _sc_item_body(item_idx, itab, out_i, idx_v, rows_v, sem):
    wid = lax.axis_index("s") * _NC + lax.axis_index("c")
    base = wid * _BPW
    pltpu.sync_copy(item_idx.at[pl.ds(base, _BPW)], idx_v)
    pltpu.async_copy(itab.at[idx_v], rows_v, sem).wait()
    pltpu.sync_copy(rows_v, out_i.at[pl.ds(base, _BPW)])


_sc_item = functools.partial(
    pl.kernel,
    mesh=plsc.VectorSubcoreMesh(core_axis_name="c", subcore_axis_name="s"),
    out_type=jax.ShapeDtypeStruct((B, K), jnp.float32),
    scratch_types=[
        pltpu.VMEM((_BPW,), jnp.int32),
        pltpu.VMEM((_BPW, K), jnp.float32),
        pltpu.SemaphoreType.DMA,
    ],
    compiler_params=pltpu.CompilerParams(use_tc_tiling_on_sc=False),
)(_sc_item_body)


_BM = 512
_BN = 4096


def _mm_body(u_ref, i_ref, o_ref):
    o_ref[...] = lax.dot_general(
        u_ref[...], i_ref[...],
        dimension_numbers=(((0,), (1,)), ((), ())),
        preferred_element_type=jnp.float32,
    )


_mm = pl.pallas_call(
    _mm_body,
    grid=(B // _BM, B // _BN),
    in_specs=[
        pl.BlockSpec((K, _BM), lambda i, j: (0, i)),
        pl.BlockSpec((_BN, K), lambda i, j: (j, 0)),
    ],
    out_specs=pl.BlockSpec((_BM, _BN), lambda i, j: (i, j)),
    out_shape=jax.ShapeDtypeStruct((B, B), jnp.float32),
)


def kernel(user, item, user_table, item_table):
    ut = _sc_user(user.astype(jnp.int32), user_table.T)
    it = _sc_item(item.astype(jnp.int32), item_table)
    return _mm(ut, it)


# speedup vs baseline: 1.3334x; 1.3334x over previous
import functools

import jax
import jax.numpy as jnp
from jax import lax
from jax.experimental import pallas as pl
from jax.experimental.pallas import tpu as pltpu
from jax.experimental.pallas import tpu_sc as plsc

B = 4096
K = 32

_info = plsc.get_sparse_core_info()
_NC, _NS = _info.num_cores, _info.num_subcores
_NW = _NC * _NS
_BPW = B // _NW
_SG = 4
_NSG = _BPW // _SG


def _splat(x, n=16):
    return jnp.full((n,), x, jnp.int32)


def _sc_gather_body(user_idx, item_idx, utab_t, itab_t, out_ut, out_it,
                    idx_u, idx_i, buf_u, buf_i, cols_u, cols_i, sem_u, sem_i):
    wid = lax.axis_index("s") * _NC + lax.axis_index("c")
    base = wid * _BPW
    pltpu.sync_copy(user_idx.at[pl.ds(base, _BPW)], idx_u)
    pltpu.sync_copy(item_idx.at[pl.ds(base, _BPW)], idx_i)
    c_lo = lax.iota(jnp.int32, 16)
    c_hi = c_lo + 16

    vregs = []
    for h in range(_BPW // 16):
        vu = idx_u[pl.ds(h * 16, 16)]
        vi = idx_i[pl.ds(h * 16, 16)]
        vregs.append((
            lax.shift_right_logical(vu, 7), lax.bitwise_and(vu, _splat(127)),
            lax.shift_right_logical(vi, 7), lax.bitwise_and(vi, _splat(127)),
        ))

    def fire(g, q):
        tu, _, ti, _ = vregs[(g * _SG) // 16]
        for k in range(_SG):
            lane = (g * _SG) % 16 + k
            offu = pl.multiple_of(tu[lane] * 128, 128)
            offi = pl.multiple_of(ti[lane] * 128, 128)
            pltpu.make_async_copy(
                utab_t.at[:, pl.ds(offu, 128)], buf_u.at[q, k], sem_u).start()
            pltpu.make_async_copy(
                itab_t.at[:, pl.ds(offi, 128)], buf_i.at[q, k], sem_i).start()

    def drain_extract(g, q):
        _, ru, _, ri = vregs[(g * _SG) // 16]
        for k in range(_SG):
            pltpu.make_async_copy(
                utab_t.at[:, pl.ds(0, 128)], buf_u.at[q, k], sem_u).wait()
            pltpu.make_async_copy(
                itab_t.at[:, pl.ds(0, 128)], buf_i.at[q, k], sem_i).wait()
        qv = _splat(q)
        for k in range(_SG):
            lane = (g * _SG) % 16 + k
            slot = _splat(k)
            jv = _splat(g * _SG + k)
            rmu = _splat(ru[lane])
            rmi = _splat(ri[lane])
            u_lo = plsc.load_gather(buf_u, [qv, slot, c_lo, rmu])
            u_hi = plsc.load_gather(buf_u, [qv, slot, c_hi, rmu])
            i_lo = plsc.load_gather(buf_i, [qv, slot, c_lo, rmi])
            i_hi = plsc.load_gather(buf_i, [qv, slot, c_hi, rmi])
            plsc.store_scatter(cols_u, [c_lo, jv], u_lo)
            plsc.store_scatter(cols_u, [c_hi, jv], u_hi)
            plsc.store_scatter(cols_i, [c_lo, jv], i_lo)
            plsc.store_scatter(cols_i, [c_hi, jv], i_hi)

    fire(0, 0)
    for g in range(_NSG):
        if g + 1 < _NSG:
            fire(g + 1, (g + 1) % 2)
        drain_extract(g, g % 2)

    pltpu.sync_copy(cols_u, out_ut.at[:, pl.ds(base, _BPW)])
    pltpu.sync_copy(cols_i, out_it.at[:, pl.ds(base, _BPW)])


_sc_gather = functools.partial(
    pl.kernel,
    mesh=plsc.VectorSubcoreMesh(core_axis_name="c", subcore_axis_name="s"),
    out_type=(
        jax.ShapeDtypeStruct((K, B), jnp.float32),
        jax.ShapeDtypeStruct((K, B), jnp.float32),
    ),
    scratch_types=[
        pltpu.VMEM((_BPW,), jnp.int32),
        pltpu.VMEM((_BPW,), jnp.int32),
        pltpu.VMEM((2, _SG, K, 128), jnp.float32),
        pltpu.VMEM((2, _SG, K, 128), jnp.float32),
        pltpu.VMEM((K, _BPW), jnp.float32),
        pltpu.VMEM((K, _BPW), jnp.float32),
        pltpu.SemaphoreType.DMA,
        pltpu.SemaphoreType.DMA,
    ],
    compiler_params=pltpu.CompilerParams(needs_layout_passes=False),
)(_sc_gather_body)


_BM = 512
_BN = 4096


def _mm_body(u_ref, i_ref, o_ref):
    o_ref[...] = lax.dot_general(
        u_ref[...], i_ref[...],
        dimension_numbers=(((0,), (0,)), ((), ())),
        preferred_element_type=jnp.float32,
    )


_mm = pl.pallas_call(
    _mm_body,
    grid=(B // _BM, B // _BN),
    in_specs=[
        pl.BlockSpec((K, _BM), lambda i, j: (0, i)),
        pl.BlockSpec((K, _BN), lambda i, j: (0, j)),
    ],
    out_specs=pl.BlockSpec((_BM, _BN), lambda i, j: (i, j)),
    out_shape=jax.ShapeDtypeStruct((B, B), jnp.float32),
)


def kernel(user, item, user_table, item_table):
    ut, it = _sc_gather(user.astype(jnp.int32), item.astype(jnp.int32),
                        user_table.T, item_table.T)
    return _mm(ut, it)
